# baseline (speedup 1.0000x reference)
import functools

import jax
import jax.numpy as jnp
from jax import lax
from jax.experimental import pallas as pl
from jax.experimental.pallas import tpu as pltpu
from jax.experimental.pallas import tpu_sc as plsc

_B, _S, _F = 1024, 64, 19
_DM = 1152
_D9 = _DM // 9
_D6 = _D9 * 6
_ROWS = _B * _S
_BLOCK_ROWS = 2048

_NC, _NS = 2, 16
_NW = _NC * _NS
_BPW = _ROWS // _NW
_CH = 128


def _sc_gather_body(dd_hbm, pl_hbm, mg_hbm, tdd_hbm, tpl_hbm, tmg_hbm,
                    g_hbm, dd_v, pl_v, mg_v, rows_v, sem):
    wid = lax.axis_index("s") * _NC + lax.axis_index("c")
    base = wid * _BPW
    pltpu.sync_copy(dd_hbm.at[pl.ds(base, _BPW)], dd_v)
    pltpu.sync_copy(pl_hbm.at[pl.ds(base, _BPW)], pl_v)
    pltpu.sync_copy(mg_hbm.at[pl.ds(base, _BPW)], mg_v)
    for t, (tab, idx_v) in enumerate(
            ((tdd_hbm, dd_v), (tpl_hbm, pl_v), (tmg_hbm, mg_v))):
        for c in range(_BPW // _CH):
            pltpu.async_copy(
                tab.at[idx_v.at[pl.ds(c * _CH, _CH)]], rows_v, sem).wait()
            pltpu.sync_copy(
                rows_v,
                g_hbm.at[pl.ds(base + c * _CH, _CH),
                         pl.ds(t * _D9, _D9)])


@functools.partial(
    pl.kernel,
    out_type=jax.ShapeDtypeStruct((_ROWS, 3 * _D9), jnp.float32),
    mesh=plsc.VectorSubcoreMesh(core_axis_name="c", subcore_axis_name="s",
                                num_cores=_NC, num_subcores=_NS),
    scratch_types=[
        pltpu.VMEM((_BPW,), jnp.int32),
        pltpu.VMEM((_BPW,), jnp.int32),
        pltpu.VMEM((_BPW,), jnp.int32),
        pltpu.VMEM((_CH, _D9), jnp.float32),
        pltpu.SemaphoreType.DMA,
    ],
)
def _sc_gather(*args):
    _sc_gather_body(*args)


def _asm_kernel(x_ref, w_ref, b_ref, pos_ref, g_ref, o_ref):
    xb = x_ref[...]
    cont = xb[:, 0:_F - 3]
    x1 = jax.lax.dot_general(
        cont, w_ref[...], (((1,), (0,)), ((), ())),
        preferred_element_type=jnp.float32) + b_ref[...]
    y = jnp.concatenate([x1, g_ref[...]], axis=1)
    y = y.reshape(_BLOCK_ROWS // _S, _S, _DM) + pos_ref[...][None]
    o_ref[...] = y.reshape(_BLOCK_ROWS, _DM)


def kernel(x, W, b, tab_dd, tab_plate, tab_magtype, tab_pos):
    x2d = x.reshape(_ROWS, _F)
    pl_i = jnp.clip(x2d[:, _F - 3].astype(jnp.int32), 0, 63)
    dd_i = jnp.clip(x2d[:, _F - 2].astype(jnp.int32), 0, 19)
    mg_i = jnp.clip(x2d[:, _F - 1].astype(jnp.int32), 0, 19)

    g = _sc_gather(dd_i, pl_i, mg_i, tab_dd, tab_plate, tab_magtype)

    b2d = b.reshape(1, _D6)
    grid = (_ROWS // _BLOCK_ROWS,)
    out = pl.pallas_call(
        _asm_kernel,
        grid=grid,
        in_specs=[
            pl.BlockSpec((_BLOCK_ROWS, _F), lambda i: (i, 0)),
            pl.BlockSpec((_F - 3, _D6), lambda i: (0, 0)),
            pl.BlockSpec((1, _D6), lambda i: (0, 0)),
            pl.BlockSpec((_S, _DM), lambda i: (0, 0)),
            pl.BlockSpec((_BLOCK_ROWS, 3 * _D9), lambda i: (i, 0)),
        ],
        out_specs=pl.BlockSpec((_BLOCK_ROWS, _DM), lambda i: (i, 0)),
        out_shape=jax.ShapeDtypeStruct((_ROWS, _DM), jnp.float32),
        compiler_params=pltpu.CompilerParams(
            dimension_semantics=("arbitrary",)),
    )(x2d, W, b2d, tab_pos, g)
    return out.reshape(_B, _S, _DM)


# speedup vs baseline: 1.1595x; 1.1595x over previous
import functools

import jax
import jax.numpy as jnp
from jax import lax
from jax.experimental import pallas as pl
from jax.experimental.pallas import tpu as pltpu
from jax.experimental.pallas import tpu_sc as plsc

_B, _S, _F = 1024, 64, 19
_DM = 1152
_D9 = _DM // 9
_D6 = _D9 * 6
_ROWS = _B * _S
_BLOCK_ROWS = 2048

_NC, _NS = 2, 16
_NW = _NC * _NS
_BPW = _ROWS // _NW
_CH = 128


def _sc_gather_body(dd_hbm, pl_hbm, mg_hbm, tdd_hbm, tpl_hbm, tmg_hbm,
                    g_hbm, dd_v, pl_v, mg_v, rows_a, rows_b, sem_a, sem_b):
    wid = lax.axis_index("s") * _NC + lax.axis_index("c")
    base = wid * _BPW
    pltpu.sync_copy(dd_hbm.at[pl.ds(base, _BPW)], dd_v)
    pltpu.sync_copy(pl_hbm.at[pl.ds(base, _BPW)], pl_v)
    pltpu.sync_copy(mg_hbm.at[pl.ds(base, _BPW)], mg_v)
    bufs = (rows_a, rows_b)
    sems = (sem_a, sem_b)
    work = [(tab, idx_v, t, c)
            for t, (tab, idx_v) in enumerate(
                ((tdd_hbm, dd_v), (tpl_hbm, pl_v), (tmg_hbm, mg_v)))
            for c in range(_BPW // _CH)]

    def fire(i):
        tab, idx_v, _, c = work[i]
        return pltpu.async_copy(
            tab.at[idx_v.at[pl.ds(c * _CH, _CH)]], bufs[i % 2], sems[i % 2])

    pending = fire(0)
    for i in range(len(work)):
        nxt = fire(i + 1) if i + 1 < len(work) else None
        pending.wait()
        _, _, t, c = work[i]
        pltpu.sync_copy(
            bufs[i % 2],
            g_hbm.at[pl.ds(base + c * _CH, _CH), pl.ds(t * _D9, _D9)])
        pending = nxt


@functools.partial(
    pl.kernel,
    out_type=jax.ShapeDtypeStruct((_ROWS, 3 * _D9), jnp.float32),
    mesh=plsc.VectorSubcoreMesh(core_axis_name="c", subcore_axis_name="s",
                                num_cores=_NC, num_subcores=_NS),
    scratch_types=[
        pltpu.VMEM((_BPW,), jnp.int32),
        pltpu.VMEM((_BPW,), jnp.int32),
        pltpu.VMEM((_BPW,), jnp.int32),
        pltpu.VMEM((_CH, _D9), jnp.float32),
        pltpu.VMEM((_CH, _D9), jnp.float32),
        pltpu.SemaphoreType.DMA,
        pltpu.SemaphoreType.DMA,
    ],
)
def _sc_gather(*args):
    _sc_gather_body(*args)


def _asm_kernel(x_ref, w_ref, b_ref, pos_ref, g_ref, o_ref):
    xb = x_ref[...]
    cont = xb[:, 0:_F - 3]
    x1 = jax.lax.dot_general(
        cont, w_ref[...], (((1,), (0,)), ((), ())),
        preferred_element_type=jnp.float32) + b_ref[...]
    y = jnp.concatenate([x1, g_ref[...]], axis=1)
    y = y.reshape(_BLOCK_ROWS // _S, _S, _DM) + pos_ref[...][None]
    o_ref[...] = y.reshape(_BLOCK_ROWS, _DM)


def kernel(x, W, b, tab_dd, tab_plate, tab_magtype, tab_pos):
    x2d = x.reshape(_ROWS, _F)
    pl_i = jnp.clip(x2d[:, _F - 3].astype(jnp.int32), 0, 63)
    dd_i = jnp.clip(x2d[:, _F - 2].astype(jnp.int32), 0, 19)
    mg_i = jnp.clip(x2d[:, _F - 1].astype(jnp.int32), 0, 19)

    g = _sc_gather(dd_i, pl_i, mg_i, tab_dd, tab_plate, tab_magtype)

    b2d = b.reshape(1, _D6)
    grid = (_ROWS // _BLOCK_ROWS,)
    out = pl.pallas_call(
        _asm_kernel,
        grid=grid,
        in_specs=[
            pl.BlockSpec((_BLOCK_ROWS, _F), lambda i: (i, 0)),
            pl.BlockSpec((_F - 3, _D6), lambda i: (0, 0)),
            pl.BlockSpec((1, _D6), lambda i: (0, 0)),
            pl.BlockSpec((_S, _DM), lambda i: (0, 0)),
            pl.BlockSpec((_BLOCK_ROWS, 3 * _D9), lambda i: (i, 0)),
        ],
        out_specs=pl.BlockSpec((_BLOCK_ROWS, _DM), lambda i: (i, 0)),
        out_shape=jax.ShapeDtypeStruct((_ROWS, _DM), jnp.float32),
        compiler_params=pltpu.CompilerParams(
            dimension_semantics=("arbitrary",)),
    )(x2d, W, b2d, tab_pos, g)
    return out.reshape(_B, _S, _DM)
